# Initial kernel scaffold; baseline (speedup 1.0000x reference)
import jax
import jax.numpy as jnp
from jax import lax
from jax.experimental import pallas as pl
from jax.experimental.pallas import tpu as pltpu
from jax.experimental.pallas import tpu_sc as plsc

B = 16
P = 4096
C = 14
L = 16
NC = 2
NS = 16
HALF = P // 2
NV = HALF // L
BIG = 1e18


def _vsqrt(x):
    i = lax.bitcast_convert_type(x, jnp.int32)
    g = lax.bitcast_convert_type((i >> 1) + 0x1FBD1DF5, jnp.float32)
    for _ in range(3):
        g = 0.5 * (g + x / g)
    return g


def _body(disp_hbm, sub_hbm, cpred_hbm, origin_hbm, tgt_hbm, out_hbm,
          px, py, pz, sx, sy, sz, dv, lv, cpv,
          hist, pbuf, qbuf, cbuf, obuf,
          sh_cent, sh_loss, sem):
    cid = lax.axis_index("c")
    sid = lax.axis_index("s")
    wid = cid * NS + sid
    b = wid >> 1
    half = wid & 1
    off = half * HALF

    copies = [
        pltpu.make_async_copy(origin_hbm.at[b, 0, pl.ds(off, HALF)], px, sem),
        pltpu.make_async_copy(origin_hbm.at[b, 1, pl.ds(off, HALF)], py, sem),
        pltpu.make_async_copy(origin_hbm.at[b, 2, pl.ds(off, HALF)], pz, sem),
        pltpu.make_async_copy(sub_hbm.at[b, 0, pl.ds(off, HALF)], sx, sem),
        pltpu.make_async_copy(sub_hbm.at[b, 1, pl.ds(off, HALF)], sy, sem),
        pltpu.make_async_copy(sub_hbm.at[b, 2, pl.ds(off, HALF)], sz, sem),
        pltpu.make_async_copy(disp_hbm.at[b, pl.ds(off, HALF)], dv, sem),
        pltpu.make_async_copy(tgt_hbm.at[b, pl.ds(off, HALF)], lv, sem),
        pltpu.make_async_copy(cpred_hbm, cpv, sem),
    ]
    for cp in copies:
        cp.start()

    zeros = jnp.zeros((L,), jnp.float32)
    def zbody(j, carry):
        hist[pl.ds(j * L, L)] = zeros
        return carry
    lax.fori_loop(0, 1024 // L, zbody, 0)

    for cp in copies:
        cp.wait()

    iota = lax.iota(jnp.int32, L)
    lane16 = iota * L

    ones = jnp.full((L,), 1.0, jnp.float32)
    def hbody(i, carry):
        s = pl.ds(i * L, L)
        lbl = lv[s]
        idx = (lbl << 4) + iota
        plsc.addupdate_scatter(hist, [idx], px[s])
        plsc.addupdate_scatter(hist, [idx + 224], py[s])
        plsc.addupdate_scatter(hist, [idx + 448], pz[s])
        plsc.addupdate_scatter(hist, [idx + 672], ones)
        return carry
    lax.fori_loop(0, NV, hbody, 0)

    sums = []
    for q in range(4):
        acc = plsc.load_gather(hist, [lane16 + q * 224])
        for k in range(1, L):
            acc = acc + plsc.load_gather(hist, [lane16 + (q * 224 + k)])
        sums.append(acc)
    pbuf[pl.ds(0, L)] = sums[0]
    pbuf[pl.ds(L, L)] = sums[1]
    pbuf[pl.ds(2 * L, L)] = sums[2]
    pbuf[pl.ds(3 * L, L)] = sums[3]

    pltpu.sync_copy(pbuf.at[pl.ds(0, 4 * L)], sh_cent.at[sid])
    plsc.subcore_barrier()
    pltpu.sync_copy(sh_cent.at[sid ^ 1], qbuf)

    msx = pbuf[pl.ds(0, L)] + qbuf[pl.ds(0, L)]
    msy = pbuf[pl.ds(L, L)] + qbuf[pl.ds(L, L)]
    msz = pbuf[pl.ds(2 * L, L)] + qbuf[pl.ds(2 * L, L)]
    mcnt = pbuf[pl.ds(3 * L, L)] + qbuf[pl.ds(3 * L, L)]
    denom = jnp.maximum(mcnt, 1.0)
    cxv = msx / denom
    cyv = msy / denom
    czv = msz / denom
    cbuf[pl.ds(0, L)] = cxv
    cbuf[pl.ds(L, L)] = cyv
    cbuf[pl.ds(2 * L, L)] = czv

    cxs = [cbuf[c] for c in range(C)]
    cys = [cbuf[L + c] for c in range(C)]
    czs = [cbuf[2 * L + c] for c in range(C)]

    def dbody(i, acc):
        s = pl.ds(i * L, L)
        x = sx[s]
        y = sy[s]
        z = sz[s]
        dp = dv[s]
        m = None
        for c in range(C):
            dx = x - cxs[c]
            dy = y - cys[c]
            dz = z - czs[c]
            d2 = dx * dx + dy * dy + dz * dz
            m = d2 if m is None else jnp.minimum(m, d2)
        dist = _vsqrt(m)
        t = dp - dist
        a = jnp.abs(t)
        r = jnp.where(a < 1.0, (0.5 * t) * t, a - 0.5)
        return acc + r
    dl_vec = lax.fori_loop(0, NV, dbody, jnp.zeros((L,), jnp.float32))
    loss = jnp.sum(dl_vec)

    lane_ok = iota < C
    cxj = jnp.where(lane_ok, cxv, BIG)
    cyj = jnp.where(lane_ok, cyv, 0.0)
    czj = jnp.where(lane_ok, czv, 0.0)
    colmin = None
    l2 = 0.0
    racc = jnp.full((L,), 1.0, jnp.float32)
    for i in range(C):
        dx = cpv[b, 0, i] - cxj
        dy = cpv[b, 1, i] - cyj
        dz = cpv[b, 2, i] - czj
        row = dx * dx + dy * dy + dz * dz
        colmin = row if colmin is None else jnp.minimum(colmin, row)
        m1 = jnp.min(row)
        l2 = l2 + m1
        is_min = row == m1
        cnt = plsc.all_reduce_population_count(is_min)
        row2 = jnp.where(is_min, BIG, row)
        m2 = jnp.min(row2)
        m2eff = jnp.where(cnt > 1, m1, m2)
        ratio = m1 / m2eff
        racc = jnp.where(iota == i, ratio, racc)
    l1 = jnp.sum(jnp.where(lane_ok, colmin, 0.0))
    sep = jnp.sum(jnp.where(lane_ok, _vsqrt(racc), 0.0))
    cham = l1 + l2 + 0.1 * sep
    loss = loss + jnp.where(half == 0, cham, 0.0)

    obuf[pl.ds(0, L)] = jnp.full((L,), loss, jnp.float32)
    pltpu.sync_copy(obuf, sh_loss.at[sid])
    plsc.subcore_barrier()

    @pl.when(sid == 0)
    def _():
        pltpu.sync_copy(sh_loss, pbuf)
        tot = pbuf[pl.ds(0, L)]
        for k in range(1, NS):
            tot = tot + pbuf[pl.ds(k * L, L)]
        obuf[pl.ds(0, L)] = tot
        pltpu.sync_copy(obuf, out_hbm.at[cid])


@jax.jit
def kernel(displacement_pred, subsampled_points, centroids_pred,
           origin_points, target):
    mesh = plsc.VectorSubcoreMesh(core_axis_name="c", subcore_axis_name="s")
    run = pl.kernel(
        _body,
        out_type=jax.ShapeDtypeStruct((NC, L), jnp.float32),
        mesh=mesh,
        scratch_types=[
            pltpu.VMEM((HALF,), jnp.float32),
            pltpu.VMEM((HALF,), jnp.float32),
            pltpu.VMEM((HALF,), jnp.float32),
            pltpu.VMEM((HALF,), jnp.float32),
            pltpu.VMEM((HALF,), jnp.float32),
            pltpu.VMEM((HALF,), jnp.float32),
            pltpu.VMEM((HALF,), jnp.float32),
            pltpu.VMEM((HALF,), jnp.int32),
            pltpu.VMEM((B, 3, C), jnp.float32),
            pltpu.VMEM((1024,), jnp.float32),
            pltpu.VMEM((NS * L,), jnp.float32),
            pltpu.VMEM((4 * L,), jnp.float32),
            pltpu.VMEM((3 * L,), jnp.float32),
            pltpu.VMEM((L,), jnp.float32),
            pltpu.VMEM_SHARED((NS, 4 * L), jnp.float32),
            pltpu.VMEM_SHARED((NS, L), jnp.float32),
            pltpu.SemaphoreType.DMA,
        ],
    )
    out = run(displacement_pred, subsampled_points, centroids_pred,
              origin_points, target.astype(jnp.int32))
    return jnp.sum(out[:, 0])


# speedup vs baseline: 7.0844x; 7.0844x over previous
import jax
import jax.numpy as jnp
from jax import lax
from jax.experimental import pallas as pl
from jax.experimental.pallas import tpu as pltpu
from jax.experimental.pallas import tpu_sc as plsc

B = 16
P = 4096
C = 14
L = 16
NC = 2
NS = 16
HALF = P // 2
NV = HALF // L
U = 4
BIG = 1e18


def _rsqrt(x):
    i = lax.bitcast_convert_type(x, jnp.int32)
    y = lax.bitcast_convert_type(0x5F3759DF - (i >> 1), jnp.float32)
    for _ in range(3):
        y = y * (1.5 - 0.5 * x * y * y)
    return y


def _tree_min(vals):
    while len(vals) > 1:
        nxt = [jnp.minimum(vals[k], vals[k + 1]) for k in range(0, len(vals) - 1, 2)]
        if len(vals) % 2:
            nxt.append(vals[-1])
        vals = nxt
    return vals[0]


def _body(disp_hbm, sub_hbm, cpred_hbm, origin_hbm, tgt_hbm, out_hbm,
          px, py, pz, sx, sy, sz, dv, lv, cpv,
          hist, pbuf, qbuf, obuf, lbig,
          sh_cent, sh_loss, sem):
    cid = lax.axis_index("c")
    sid = lax.axis_index("s")
    wid = cid * NS + sid
    b = wid >> 1
    half = wid & 1
    off = half * HALF

    b1 = pl.ds(b, 1)
    sl = pl.ds(off, HALF)
    copies = [
        pltpu.make_async_copy(origin_hbm.at[b1, pl.ds(0, 1), sl], px, sem),
        pltpu.make_async_copy(origin_hbm.at[b1, pl.ds(1, 1), sl], py, sem),
        pltpu.make_async_copy(origin_hbm.at[b1, pl.ds(2, 1), sl], pz, sem),
        pltpu.make_async_copy(sub_hbm.at[b1, pl.ds(0, 1), sl], sx, sem),
        pltpu.make_async_copy(sub_hbm.at[b1, pl.ds(1, 1), sl], sy, sem),
        pltpu.make_async_copy(sub_hbm.at[b1, pl.ds(2, 1), sl], sz, sem),
        pltpu.make_async_copy(disp_hbm.at[b1, sl], dv, sem),
        pltpu.make_async_copy(tgt_hbm.at[b1, sl], lv, sem),
        pltpu.make_async_copy(cpred_hbm, cpv, sem),
    ]
    for cp in copies:
        cp.start()

    zeros = jnp.zeros((L,), jnp.float32)
    def zbody(j, carry):
        hist[pl.ds(j * L, L)] = zeros
        return carry
    lax.fori_loop(0, 1024 // L, zbody, 0)

    for cp in copies:
        cp.wait()

    pxr = px.at[0, 0]
    pyr = py.at[0, 0]
    pzr = pz.at[0, 0]
    sxr = sx.at[0, 0]
    syr = sy.at[0, 0]
    szr = sz.at[0, 0]
    dvr = dv.at[0]
    lvr = lv.at[0]

    iota = lax.iota(jnp.int32, L)
    lane16 = iota * L

    ones = jnp.full((L,), 1.0, jnp.float32)
    def hbody(i, carry):
        for u in range(2):
            s = pl.ds((2 * i + u) * L, L)
            lbl = lvr[s]
            idx = (lbl << 4) + iota
            plsc.addupdate_scatter(hist, [idx], pxr[s])
            plsc.addupdate_scatter(hist, [idx + 224], pyr[s])
            plsc.addupdate_scatter(hist, [idx + 448], pzr[s])
            plsc.addupdate_scatter(hist, [idx + 672], ones)
        return carry
    lax.fori_loop(0, NV // 2, hbody, 0)

    sums = []
    for q in range(4):
        acc = plsc.load_gather(hist, [lane16 + q * 224])
        for k in range(1, L):
            acc = acc + plsc.load_gather(hist, [lane16 + (q * 224 + k)])
        sums.append(acc)
    pbuf[pl.ds(0, L)] = sums[0]
    pbuf[pl.ds(L, L)] = sums[1]
    pbuf[pl.ds(2 * L, L)] = sums[2]
    pbuf[pl.ds(3 * L, L)] = sums[3]

    pltpu.sync_copy(pbuf, sh_cent.at[pl.ds(sid * (4 * L), 4 * L)])
    plsc.subcore_barrier()
    pltpu.sync_copy(sh_cent.at[pl.ds((sid ^ 1) * (4 * L), 4 * L)], qbuf)

    msx = pbuf[pl.ds(0, L)] + qbuf[pl.ds(0, L)]
    msy = pbuf[pl.ds(L, L)] + qbuf[pl.ds(L, L)]
    msz = pbuf[pl.ds(2 * L, L)] + qbuf[pl.ds(2 * L, L)]
    mcnt = pbuf[pl.ds(3 * L, L)] + qbuf[pl.ds(3 * L, L)]
    denom = jnp.maximum(mcnt, 1.0)
    cxv = msx / denom
    cyv = msy / denom
    czv = msz / denom
    c2v = cxv * cxv + cyv * cyv + czv * czv

    cxs = [cxv[c] for c in range(C)]
    cys = [cyv[c] for c in range(C)]
    czs = [czv[c] for c in range(C)]
    c2s = [c2v[c] for c in range(C)]

    def dbody(i, accs):
        res = []
        for u in range(U):
            s = pl.ds((U * i + u) * L, L)
            x = sxr[s]
            y = syr[s]
            z = szr[s]
            dp = dvr[s]
            s2 = x * x + y * y + z * z
            d2s = []
            for c in range(C):
                t = x * cxs[c] + y * cys[c] + z * czs[c]
                d2s.append(s2 + (c2s[c] - (t + t)))
            m = jnp.maximum(_tree_min(d2s), 0.0)
            dist = m * _rsqrt(m)
            t0 = dp - dist
            a = jnp.abs(t0)
            res.append(jnp.where(a < 1.0, (0.5 * t0) * t0, a - 0.5))
        return tuple(accs[u] + res[u] for u in range(U))
    acc0 = tuple(jnp.zeros((L,), jnp.float32) for _ in range(U))
    accs = lax.fori_loop(0, NV // U, dbody, acc0)
    loss = jnp.sum(accs[0] + accs[1] + accs[2] + accs[3])

    lane_ok = iota < C
    cxj = jnp.where(lane_ok, cxv, BIG)
    cyj = jnp.where(lane_ok, cyv, 0.0)
    czj = jnp.where(lane_ok, czv, 0.0)
    colmin = None
    l2 = 0.0
    racc = jnp.full((L,), 1.0, jnp.float32)
    cpx = cpv[pl.ds(b * (3 * L), L)]
    cpy = cpv[pl.ds(b * (3 * L) + L, L)]
    cpz = cpv[pl.ds(b * (3 * L) + 2 * L, L)]
    for i in range(C):
        dx = cpx[i] - cxj
        dy = cpy[i] - cyj
        dz = cpz[i] - czj
        row = dx * dx + dy * dy + dz * dz
        colmin = row if colmin is None else jnp.minimum(colmin, row)
        m1 = jnp.min(row)
        l2 = l2 + m1
        is_min = row == m1
        cnt = plsc.all_reduce_population_count(is_min)
        row2 = jnp.where(is_min, BIG, row)
        m2 = jnp.min(row2)
        m2eff = jnp.where(cnt > 1, m1, m2)
        ratio = m1 / m2eff
        racc = jnp.where(iota == i, ratio, racc)
    l1 = jnp.sum(jnp.where(lane_ok, colmin, 0.0))
    sepv = racc * _rsqrt(racc)
    sep = jnp.sum(jnp.where(lane_ok, sepv, 0.0))
    cham = l1 + l2 + 0.1 * sep
    loss = loss + jnp.where(half == 0, cham, 0.0)

    obuf[pl.ds(0, L)] = jnp.full((L,), loss, jnp.float32)
    pltpu.sync_copy(obuf, sh_loss.at[pl.ds(sid * L, L)])
    plsc.subcore_barrier()

    @pl.when(sid == 0)
    def _():
        pltpu.sync_copy(sh_loss, lbig)
        tot = lbig[pl.ds(0, L)]
        for k in range(1, NS):
            tot = tot + lbig[pl.ds(k * L, L)]
        obuf[pl.ds(0, L)] = tot
        pltpu.sync_copy(obuf, out_hbm.at[cid])


@jax.jit
def kernel(displacement_pred, subsampled_points, centroids_pred,
           origin_points, target):
    mesh = plsc.VectorSubcoreMesh(core_axis_name="c", subcore_axis_name="s")
    run = pl.kernel(
        _body,
        out_type=jax.ShapeDtypeStruct((NC, L), jnp.float32),
        mesh=mesh,
        compiler_params=pltpu.CompilerParams(needs_layout_passes=False),
        scratch_types=[
            pltpu.VMEM((1, 1, HALF), jnp.float32),
            pltpu.VMEM((1, 1, HALF), jnp.float32),
            pltpu.VMEM((1, 1, HALF), jnp.float32),
            pltpu.VMEM((1, 1, HALF), jnp.float32),
            pltpu.VMEM((1, 1, HALF), jnp.float32),
            pltpu.VMEM((1, 1, HALF), jnp.float32),
            pltpu.VMEM((1, HALF), jnp.float32),
            pltpu.VMEM((1, HALF), jnp.int32),
            pltpu.VMEM((B * 3 * L,), jnp.float32),
            pltpu.VMEM((1024,), jnp.float32),
            pltpu.VMEM((4 * L,), jnp.float32),
            pltpu.VMEM((4 * L,), jnp.float32),
            pltpu.VMEM((L,), jnp.float32),
            pltpu.VMEM((NS * L,), jnp.float32),
            pltpu.VMEM_SHARED((NS * 4 * L,), jnp.float32),
            pltpu.VMEM_SHARED((NS * L,), jnp.float32),
            pltpu.SemaphoreType.DMA,
        ],
    )
    cpred_pad = jnp.pad(centroids_pred, ((0, 0), (0, 0), (0, L - C))).reshape(-1)
    out = run(displacement_pred, subsampled_points, cpred_pad,
              origin_points, target.astype(jnp.int32))
    return jnp.sum(out[:, 0])
